# baseline (device time: 27675 ns/iter reference)
import jax
import jax.numpy as jnp
from jax import lax
from jax.experimental import pallas as pl
from jax.experimental.pallas import tpu as pltpu

N_DEV = 4


def kernel(A, B):
    m, k = A.shape
    _, n = B.shape

    def body(a_ref, b_ref, out_ref, comm_ref, send_sems, recv_sems):
        my = lax.axis_index("i")
        left = (my - 1) % N_DEV
        right = (my + 1) % N_DEV

        partial = jnp.dot(
            a_ref[...].astype(jnp.bfloat16),
            b_ref[...].astype(jnp.bfloat16),
            preferred_element_type=jnp.float32,
        )
        out_ref[...] = partial
        comm_ref[0] = partial.astype(jnp.bfloat16)

        barrier_sem = pltpu.get_barrier_semaphore()
        for nbr in (left, right):
            pl.semaphore_signal(
                barrier_sem, inc=1,
                device_id=(nbr,), device_id_type=pl.DeviceIdType.MESH,
            )
        pl.semaphore_wait(barrier_sem, 2)

        for h in range(N_DEV - 1):
            rdma = pltpu.make_async_remote_copy(
                src_ref=comm_ref.at[h],
                dst_ref=comm_ref.at[h + 1],
                send_sem=send_sems.at[h],
                recv_sem=recv_sems.at[h],
                device_id=(right,),
                device_id_type=pl.DeviceIdType.MESH,
            )
            rdma.start()
            rdma.wait()
            out_ref[...] = out_ref[...] + comm_ref[h + 1].astype(jnp.float32)

        out_ref[...] = jnp.maximum(out_ref[...], 0.0)

    return pl.pallas_call(
        body,
        out_shape=jax.ShapeDtypeStruct((m, n), jnp.float32),
        in_specs=[
            pl.BlockSpec(memory_space=pltpu.VMEM),
            pl.BlockSpec(memory_space=pltpu.VMEM),
        ],
        out_specs=pl.BlockSpec(memory_space=pltpu.VMEM),
        scratch_shapes=[
            pltpu.VMEM((N_DEV, m, n), jnp.bfloat16),
            pltpu.SemaphoreType.DMA((N_DEV - 1,)),
            pltpu.SemaphoreType.DMA((N_DEV - 1,)),
        ],
        compiler_params=pltpu.CompilerParams(collective_id=0),
    )(A, B)


# device time: 15967 ns/iter; 1.7333x vs baseline; 1.7333x over previous
import jax
import jax.numpy as jnp
from jax import lax
from jax.experimental import pallas as pl
from jax.experimental.pallas import tpu as pltpu

N_DEV = 4


def kernel(A, B):
    m, k = A.shape
    _, n = B.shape
    ch = m // N_DEV

    def body(a_ref, b_ref, out_ref, send_buf, rs_buf, red_buf, ag_src,
             ag_buf, rs_send_sems, rs_sems, ag_send_sems, ag_sems):
        my = lax.axis_index("i")

        barrier_sem = pltpu.get_barrier_semaphore()
        for h in range(N_DEV - 1):
            pl.semaphore_signal(
                barrier_sem, inc=1,
                device_id=((my + 1 + h) % N_DEV,),
                device_id_type=pl.DeviceIdType.MESH,
            )

        partial = jnp.dot(
            a_ref[...].astype(jnp.bfloat16),
            b_ref[...].astype(jnp.bfloat16),
            preferred_element_type=jnp.float32,
        )
        send_buf[...] = partial.astype(jnp.bfloat16)
        out_ref[...] = partial

        pl.semaphore_wait(barrier_sem, N_DEV - 1)

        rs_rdmas = []
        for h in range(N_DEV - 1):
            dest = (my + 1 + h) % N_DEV
            rdma = pltpu.make_async_remote_copy(
                src_ref=send_buf.at[pl.ds(dest * ch, ch), :],
                dst_ref=rs_buf.at[2 - h],
                send_sem=rs_send_sems.at[h],
                recv_sem=rs_sems.at[2 - h],
                device_id=(dest,),
                device_id_type=pl.DeviceIdType.MESH,
            )
            rdma.start()
            rs_rdmas.append(rdma)

        red_buf[...] = out_ref[pl.ds(my * ch, ch), :]
        for j in range(N_DEV - 1):
            recv = pltpu.make_async_remote_copy(
                src_ref=rs_buf.at[j],
                dst_ref=rs_buf.at[j],
                send_sem=rs_send_sems.at[0],
                recv_sem=rs_sems.at[j],
                device_id=(my,),
                device_id_type=pl.DeviceIdType.MESH,
            )
            recv.wait_recv()
            red_buf[...] = red_buf[...] + rs_buf[j].astype(jnp.float32)

        red = jnp.maximum(red_buf[...], 0.0)
        ag_src[...] = red.astype(jnp.bfloat16)

        ag_rdmas = []
        for h in range(N_DEV - 1):
            dest = (my + 1 + h) % N_DEV
            rdma = pltpu.make_async_remote_copy(
                src_ref=ag_src,
                dst_ref=ag_buf.at[2 - h],
                send_sem=ag_send_sems.at[h],
                recv_sem=ag_sems.at[2 - h],
                device_id=(dest,),
                device_id_type=pl.DeviceIdType.MESH,
            )
            rdma.start()
            ag_rdmas.append(rdma)

        out_ref[pl.ds(my * ch, ch), :] = red
        for j in range(N_DEV - 1):
            recv = pltpu.make_async_remote_copy(
                src_ref=ag_buf.at[j],
                dst_ref=ag_buf.at[j],
                send_sem=ag_send_sems.at[0],
                recv_sem=ag_sems.at[j],
                device_id=(my,),
                device_id_type=pl.DeviceIdType.MESH,
            )
            recv.wait_recv()
            src_chip = (my + 1 + j) % N_DEV
            out_ref[pl.ds(src_chip * ch, ch), :] = (
                ag_buf[j].astype(jnp.float32)
            )

        for rdma in rs_rdmas + ag_rdmas:
            rdma.wait_send()

    return pl.pallas_call(
        body,
        out_shape=jax.ShapeDtypeStruct((m, n), jnp.float32),
        in_specs=[
            pl.BlockSpec(memory_space=pltpu.VMEM),
            pl.BlockSpec(memory_space=pltpu.VMEM),
        ],
        out_specs=pl.BlockSpec(memory_space=pltpu.VMEM),
        scratch_shapes=[
            pltpu.VMEM((m, n), jnp.bfloat16),
            pltpu.VMEM((N_DEV - 1, ch, n), jnp.bfloat16),
            pltpu.VMEM((ch, n), jnp.float32),
            pltpu.VMEM((ch, n), jnp.bfloat16),
            pltpu.VMEM((N_DEV - 1, ch, n), jnp.bfloat16),
            pltpu.SemaphoreType.DMA((N_DEV - 1,)),
            pltpu.SemaphoreType.DMA((N_DEV - 1,)),
            pltpu.SemaphoreType.DMA((N_DEV - 1,)),
            pltpu.SemaphoreType.DMA((N_DEV - 1,)),
        ],
        compiler_params=pltpu.CompilerParams(collective_id=0),
    )(A, B)
